# baseline (device time: 382778 ns/iter reference)
import jax
import jax.numpy as jnp
from jax import lax
from jax.experimental import pallas as pl
from jax.experimental.pallas import tpu as pltpu

N_DEV = 16
S = 512
D = 1024
HEADS = 8
DH = 128
SCALE = 0.08838834764831843


def kernel(x, Wq, Wo, Wk, Wv):
    def body(x_ref, wq_ref, wo_ref, wk_ref, wv_ref, out_ref,
             xL, aL, xR, aR, x_own,
             sxL, rxL, saL, raL, sxR, rxR, saR, raR,
             creditL, creditR):
        my = lax.axis_index("i")
        RING = [0, 1, 5, 9, 13, 14, 10, 6, 2, 3, 7, 11, 15, 12, 8, 4]
        k = sum(j * (my == RING[j]) for j in range(N_DEV))
        left = sum(RING[(j - 1) % N_DEV] * (my == RING[j])
                   for j in range(N_DEV))
        right = sum(RING[(j + 1) % N_DEV] * (my == RING[j])
                    for j in range(N_DEV))
        my_odd = lax.rem(k, 2) == 1

        def f_partial(xc):
            xc = xc.astype(jnp.float32)
            q = jnp.dot(xc, wq_ref[:, :], preferred_element_type=jnp.float32)
            k = jnp.dot(xc, wk_ref[:, :], preferred_element_type=jnp.float32)
            v = jnp.dot(xc, wv_ref[:, :], preferred_element_type=jnp.float32)
            outs = []
            for j in range(HEADS):
                sl = slice(j * DH, (j + 1) * DH)
                s = lax.dot_general(
                    q[:, sl], k[:, sl],
                    (((1,), (1,)), ((), ())),
                    preferred_element_type=jnp.float32,
                ) * SCALE
                m = jnp.max(s, axis=1, keepdims=True)
                p = jnp.exp(s - m)
                l = jnp.sum(p, axis=1, keepdims=True)
                o = jnp.dot(p, v[:, sl], preferred_element_type=jnp.float32) / l
                outs.append(o)
            attn = jnp.concatenate(outs, axis=1)
            return jnp.dot(attn, wo_ref[:, :], preferred_element_type=jnp.float32)

        def make(src, dst, ssem, rsem, dev):
            return pltpu.make_async_remote_copy(
                src_ref=src, dst_ref=dst, send_sem=ssem, recv_sem=rsem,
                device_id=(dev,), device_id_type=pl.DeviceIdType.MESH,
            )

        def ring_block(h, xc, ac, sxr, rxr, sar, rar, cred, out_nbr, in_nbr):
            t = lax.div(h, 2)
            slot = lax.rem(t, 2)
            nxt = lax.rem(t + 1, 2)
            dst = lax.select(lax.rem(h, 2) == 0, slot, nxt)

            @pl.when((h >= 3) & (h <= 14))
            def _():
                pl.semaphore_wait(cred, 1)

            make(xc.at[slot], xc.at[slot], sxr.at[slot], rxr.at[slot],
                 in_nbr).wait_recv()

            dxh = make(xc.at[slot], xc.at[dst], sxr.at[dst], rxr.at[dst],
                       out_nbr)

            @pl.when(h <= 13)
            def _():
                dxh.start()

            part = f_partial(xc[slot])

            @pl.when(h >= 1)
            def _():
                make(ac.at[slot], ac.at[slot], sar.at[slot], rar.at[slot],
                     in_nbr).wait_recv()

            ac[slot] = (ac[slot].astype(jnp.float32) + part).astype(jnp.bfloat16)

            da = make(ac.at[slot], ac.at[dst], sar.at[dst], rar.at[dst],
                      out_nbr)
            da.start()

            @pl.when(h <= 13)
            def _():
                dxh.wait_send()

            da.wait_send()

            @pl.when(h <= 11)
            def _():
                pl.semaphore_signal(
                    cred, inc=1,
                    device_id=(in_nbr,), device_id_type=pl.DeviceIdType.MESH,
                )

        x_own[...] = x_ref[0].astype(jnp.bfloat16)

        @pl.when(my_odd)
        def _():
            aR[0] = jnp.zeros((S, D), jnp.bfloat16)
            pre = make(x_own, xL.at[0], sxL.at[0], rxL.at[0], left)
            pre.start()
            pre.wait_send()

        @pl.when(jnp.logical_not(my_odd))
        def _():
            aL[0] = jnp.zeros((S, D), jnp.bfloat16)
            pre = make(x_own, xR.at[0], sxR.at[0], rxR.at[0], right)
            pre.start()
            pre.wait_send()

        def step(h, carry):
            is_L = lax.rem(k + h, 2) == 0

            @pl.when(is_L)
            def _():
                ring_block(h, xL, aL, sxL, rxL, saL, raL, creditL,
                           left, right)

            @pl.when(jnp.logical_not(is_L))
            def _():
                ring_block(h, xR, aR, sxR, rxR, saR, raR, creditR,
                           right, left)

            return carry

        lax.fori_loop(0, N_DEV - 1, step, None)

        part = f_partial(x_ref[0])

        @pl.when(my_odd)
        def _():
            make(aL.at[1], aL.at[1], saL.at[1], raL.at[1], right).wait_recv()
            out_ref[0] = aL[1].astype(jnp.float32) + part

        @pl.when(jnp.logical_not(my_odd))
        def _():
            make(aR.at[1], aR.at[1], saR.at[1], raR.at[1], left).wait_recv()
            out_ref[0] = aR[1].astype(jnp.float32) + part

    return pl.pallas_call(
        body,
        out_shape=jax.ShapeDtypeStruct((1, S, D), jnp.float32),
        in_specs=[pl.BlockSpec(memory_space=pltpu.VMEM)] * 5,
        out_specs=pl.BlockSpec(memory_space=pltpu.VMEM),
        scratch_shapes=[
            pltpu.VMEM((2, S, D), jnp.bfloat16),
            pltpu.VMEM((2, S, D), jnp.bfloat16),
            pltpu.VMEM((2, S, D), jnp.bfloat16),
            pltpu.VMEM((2, S, D), jnp.bfloat16),
            pltpu.VMEM((S, D), jnp.bfloat16),
            pltpu.SemaphoreType.DMA((2,)),
            pltpu.SemaphoreType.DMA((2,)),
            pltpu.SemaphoreType.DMA((2,)),
            pltpu.SemaphoreType.DMA((2,)),
            pltpu.SemaphoreType.DMA((2,)),
            pltpu.SemaphoreType.DMA((2,)),
            pltpu.SemaphoreType.DMA((2,)),
            pltpu.SemaphoreType.DMA((2,)),
            pltpu.SemaphoreType.REGULAR,
            pltpu.SemaphoreType.REGULAR,
        ],
    )(x, Wq, Wo, Wk, Wv)


# device time: 246769 ns/iter; 1.5512x vs baseline; 1.5512x over previous
import jax
import jax.numpy as jnp
from jax import lax
from jax.experimental import pallas as pl
from jax.experimental.pallas import tpu as pltpu

N_DEV = 16
S = 512
D = 1024
HEADS = 8
DH = 128
SCALE = 0.08838834764831843


def kernel(x, Wq, Wo, Wk, Wv):
    def body(x_ref, wq_ref, wo_ref, wk_ref, wv_ref, out_ref,
             xL, aL, xR, aR, x_own,
             sxL, rxL, saL, raL, sxR, rxR, saR, raR,
             creditL, creditR):
        my = lax.axis_index("i")
        RING = [0, 1, 5, 9, 13, 14, 10, 6, 2, 3, 7, 11, 15, 12, 8, 4]
        k = sum(j * (my == RING[j]) for j in range(N_DEV))
        left = sum(RING[(j - 1) % N_DEV] * (my == RING[j])
                   for j in range(N_DEV))
        right = sum(RING[(j + 1) % N_DEV] * (my == RING[j])
                    for j in range(N_DEV))
        my_odd = lax.rem(k, 2) == 1

        def f_partial(xc):
            xc = xc.astype(jnp.float32)
            q = jnp.dot(xc, wq_ref[:, :], preferred_element_type=jnp.float32)
            k = jnp.dot(xc, wk_ref[:, :], preferred_element_type=jnp.float32)
            v = jnp.dot(xc, wv_ref[:, :], preferred_element_type=jnp.float32)
            outs = []
            for j in range(HEADS):
                sl = slice(j * DH, (j + 1) * DH)
                s = lax.dot_general(
                    q[:, sl], k[:, sl],
                    (((1,), (1,)), ((), ())),
                    preferred_element_type=jnp.float32,
                ) * SCALE
                m = jnp.max(s, axis=1, keepdims=True)
                p = jnp.exp(s - m)
                l = jnp.sum(p, axis=1, keepdims=True)
                o = jnp.dot(p, v[:, sl], preferred_element_type=jnp.float32) / l
                outs.append(o)
            attn = jnp.concatenate(outs, axis=1)
            return jnp.dot(attn, wo_ref[:, :], preferred_element_type=jnp.float32)

        def make(src, dst, ssem, rsem, dev):
            return pltpu.make_async_remote_copy(
                src_ref=src, dst_ref=dst, send_sem=ssem, recv_sem=rsem,
                device_id=(dev,), device_id_type=pl.DeviceIdType.MESH,
            )

        def ring_block(h, xc, ac, sxr, rxr, sar, rar, cred, out_nbr, in_nbr):
            t = lax.div(h, 2)
            slot = lax.rem(t, 2)
            nxt = lax.rem(t + 1, 2)
            dst = lax.select(lax.rem(h, 2) == 0, slot, nxt)

            dst_prev = lax.select(lax.rem(h, 2) == 0, nxt, slot)

            @pl.when(h >= 2)
            def _():
                make(xc.at[dst_prev], xc.at[dst_prev], sxr.at[dst_prev],
                     rxr.at[dst_prev], in_nbr).wait_send()
                make(ac.at[dst_prev], ac.at[dst_prev], sar.at[dst_prev],
                     rar.at[dst_prev], in_nbr).wait_send()

            @pl.when((h >= 2) & (h <= 13))
            def _():
                pl.semaphore_signal(
                    cred, inc=1,
                    device_id=(in_nbr,), device_id_type=pl.DeviceIdType.MESH,
                )

            @pl.when((h >= 3) & (h <= 14))
            def _():
                pl.semaphore_wait(cred, 1)

            make(xc.at[slot], xc.at[slot], sxr.at[slot], rxr.at[slot],
                 in_nbr).wait_recv()

            dxh = make(xc.at[slot], xc.at[dst], sxr.at[dst], rxr.at[dst],
                       out_nbr)

            @pl.when(h <= 13)
            def _():
                dxh.start()

            part = f_partial(xc[slot])

            @pl.when(h >= 1)
            def _():
                make(ac.at[slot], ac.at[slot], sar.at[slot], rar.at[slot],
                     in_nbr).wait_recv()

            ac[slot] = (ac[slot].astype(jnp.float32) + part).astype(jnp.bfloat16)

            da = make(ac.at[slot], ac.at[dst], sar.at[dst], rar.at[dst],
                      out_nbr)
            da.start()


        x_own[...] = x_ref[0].astype(jnp.bfloat16)

        @pl.when(my_odd)
        def _():
            aR[0] = jnp.zeros((S, D), jnp.bfloat16)
            pre = make(x_own, xL.at[0], sxL.at[0], rxL.at[0], left)
            pre.start()
            pre.wait_send()

        @pl.when(jnp.logical_not(my_odd))
        def _():
            aL[0] = jnp.zeros((S, D), jnp.bfloat16)
            pre = make(x_own, xR.at[0], sxR.at[0], rxR.at[0], right)
            pre.start()
            pre.wait_send()

        def step(h, carry):
            is_L = lax.rem(k + h, 2) == 0

            @pl.when(is_L)
            def _():
                ring_block(h, xL, aL, sxL, rxL, saL, raL, creditL,
                           left, right)

            @pl.when(jnp.logical_not(is_L))
            def _():
                ring_block(h, xR, aR, sxR, rxR, saR, raR, creditR,
                           right, left)

            return carry

        lax.fori_loop(0, N_DEV - 1, step, None)

        @pl.when(my_odd)
        def _():
            make(xL.at[1], xL.at[1], sxL.at[1], rxL.at[1], right).wait_send()
            make(aL.at[1], aL.at[1], saL.at[1], raL.at[1], right).wait_send()
            make(aR.at[1], aR.at[1], saR.at[1], raR.at[1], left).wait_send()

        @pl.when(jnp.logical_not(my_odd))
        def _():
            make(xR.at[1], xR.at[1], sxR.at[1], rxR.at[1], left).wait_send()
            make(aR.at[1], aR.at[1], saR.at[1], raR.at[1], left).wait_send()
            make(aL.at[1], aL.at[1], saL.at[1], raL.at[1], right).wait_send()

        part = f_partial(x_ref[0])

        @pl.when(my_odd)
        def _():
            make(aL.at[1], aL.at[1], saL.at[1], raL.at[1], right).wait_recv()
            out_ref[0] = aL[1].astype(jnp.float32) + part

        @pl.when(jnp.logical_not(my_odd))
        def _():
            make(aR.at[1], aR.at[1], saR.at[1], raR.at[1], left).wait_recv()
            out_ref[0] = aR[1].astype(jnp.float32) + part

    return pl.pallas_call(
        body,
        out_shape=jax.ShapeDtypeStruct((1, S, D), jnp.float32),
        in_specs=[pl.BlockSpec(memory_space=pltpu.VMEM)] * 5,
        out_specs=pl.BlockSpec(memory_space=pltpu.VMEM),
        scratch_shapes=[
            pltpu.VMEM((2, S, D), jnp.bfloat16),
            pltpu.VMEM((2, S, D), jnp.bfloat16),
            pltpu.VMEM((2, S, D), jnp.bfloat16),
            pltpu.VMEM((2, S, D), jnp.bfloat16),
            pltpu.VMEM((S, D), jnp.bfloat16),
            pltpu.SemaphoreType.DMA((2,)),
            pltpu.SemaphoreType.DMA((2,)),
            pltpu.SemaphoreType.DMA((2,)),
            pltpu.SemaphoreType.DMA((2,)),
            pltpu.SemaphoreType.DMA((2,)),
            pltpu.SemaphoreType.DMA((2,)),
            pltpu.SemaphoreType.DMA((2,)),
            pltpu.SemaphoreType.DMA((2,)),
            pltpu.SemaphoreType.REGULAR,
            pltpu.SemaphoreType.REGULAR,
        ],
    )(x, Wq, Wo, Wk, Wv)
